# baseline (speedup 1.0000x reference)
import functools

import jax
import jax.numpy as jnp
from jax import lax
from jax.experimental import pallas as pl
from jax.experimental.pallas import tpu as pltpu
from jax.experimental.pallas import tpu_sc as plsc

H, W = 512, 512
HW = H * W
B = 128
NC, NS, LANES = 2, 16, 16
NW = NC * NS
UNR = 8
NACC = 2
SPB = 4
NSTEP = B // (2 * SPB)
INT_MAX = 2**31 - 1


def _tc_block_argmax(buf, out_ref, base):
    pre = (lax.broadcasted_iota(jnp.int32, (8, W), 0) * W
           + lax.broadcasted_iota(jnp.int32, (8, W), 1))
    neg = jnp.full((8, W), -jnp.inf, jnp.float32)
    zer = jnp.zeros((8, W), jnp.int32)
    ngrp = H // 8

    for g in range(SPB):
        def body(k, carry, g=g):
            acc = list(carry)
            for t in range(UNR):
                kt = k * UNR + t
                v = buf[g, pl.ds(kt * 8, 8), :]
                p = t % NACC
                av, ai = acc[2 * p], acc[2 * p + 1]
                m = v > av
                acc[2 * p] = jnp.where(m, v, av)
                acc[2 * p + 1] = jnp.where(
                    m, jnp.full((8, W), kt, jnp.int32), ai)
            return tuple(acc)

        acc = list(lax.fori_loop(0, ngrp // UNR, body, (neg, zer) * NACC))
        pairs = [(acc[2 * p], acc[2 * p + 1] * (8 * W) + pre)
                 for p in range(NACC)]
        while len(pairs) > 1:
            out = []
            for q in range(0, len(pairs), 2):
                (av0, ai0), (av1, ai1) = pairs[q], pairs[q + 1]
                better = (av1 > av0) | ((av1 == av0) & (ai1 < ai0))
                out.append((jnp.where(better, av1, av0),
                            jnp.where(better, ai1, ai0)))
            pairs = out
        av, ai = pairs[0]
        m = jnp.max(av)
        cand = jnp.where(av == m, ai, jnp.int32(INT_MAX))
        out_ref[base + g] = jnp.broadcast_to(jnp.min(cand), (1, 128))


def _tc_argmax_kernel(x_hbm, out_ref, buf0, buf1, sem0, sem1):
    i = pl.program_id(0)

    def copy(blk, buf, sem):
        return pltpu.make_async_copy(
            x_hbm.at[pl.ds(blk * SPB, SPB)], buf, sem)

    @pl.when(i == 0)
    def _():
        copy(0, buf0, sem0).start()

    copy(2 * i + 1, buf1, sem1).start()
    copy(2 * i, buf0, sem0).wait()
    _tc_block_argmax(buf0, out_ref, 0)

    @pl.when(i < NSTEP - 1)
    def _():
        copy(2 * i + 2, buf0, sem0).start()

    copy(2 * i + 1, buf1, sem1).wait()
    _tc_block_argmax(buf1, out_ref, SPB)


def _sc_gather_kernel(idx_hbm, gx, gy, outx, outy, idxv, gatv, sem):
    cid = lax.axis_index("c")
    sid = lax.axis_index("s")
    wid = sid * NC + cid

    @pl.when(wid == 0)
    def _():
        pltpu.sync_copy(idx_hbm, idxv)
        pltpu.make_async_copy(gx.at[idxv], gatv, sem).start()
        pltpu.make_async_copy(gx.at[idxv], gatv, sem).wait()
        pltpu.sync_copy(gatv, outx)
        pltpu.make_async_copy(gy.at[idxv], gatv, sem).start()
        pltpu.make_async_copy(gy.at[idxv], gatv, sem).wait()
        pltpu.sync_copy(gatv, outy)


@jax.jit
def kernel(input, grid_x, grid_y):
    xr = input.reshape(B, H, W)
    gx1 = grid_x.reshape(HW)
    gy1 = grid_y.reshape(HW)

    idx3 = pl.pallas_call(
        _tc_argmax_kernel,
        out_shape=jax.ShapeDtypeStruct((B, 1, 128), jnp.int32),
        grid=(NSTEP,),
        in_specs=[pl.BlockSpec(memory_space=pl.ANY)],
        out_specs=pl.BlockSpec((2 * SPB, 1, 128), lambda i: (i, 0, 0)),
        scratch_shapes=[
            pltpu.VMEM((SPB, H, W), jnp.float32),
            pltpu.VMEM((SPB, H, W), jnp.float32),
            pltpu.SemaphoreType.DMA,
            pltpu.SemaphoreType.DMA,
        ],
    )(xr)
    idx = idx3[:, 0, 0]

    gather = functools.partial(
        pl.kernel,
        out_type=[
            jax.ShapeDtypeStruct((B,), jnp.float32),
            jax.ShapeDtypeStruct((B,), jnp.float32),
        ],
        mesh=plsc.VectorSubcoreMesh(core_axis_name="c", subcore_axis_name="s"),
        scratch_types=[
            pltpu.VMEM((B,), jnp.int32),
            pltpu.VMEM((B,), jnp.float32),
            pltpu.SemaphoreType.DMA,
        ],
    )(_sc_gather_kernel)
    x, y = gather(idx, gx1, gy1)
    return jnp.concatenate((x.reshape(B, 1), y.reshape(B, 1)), axis=1)
